# baseline (speedup 1.0000x reference)
import jax
import jax.numpy as jnp
from jax import lax
from jax.experimental import pallas as pl
from jax.experimental.pallas import tpu as pltpu

V, E, H, B = 50000, 512, 512, 128
TENC, TCTX, OOV = 128, 128, 50
BB = 64
NC = B // BB
TV = 512
NV = (V + TV - 1) // TV
WTOT = NV * TV
VOUT = V + OOV
NEG = -1e30


def _stage1_kernel(ids_ref, h0_ref, c0_ref, ctxvar_ref,
                   wih_ref, whh_ref, bihh_ref, attnw_ref, attnb_ref,
                   cattnw_ref, cattnb_ref, genw_ref, genb_ref,
                   emb_hbm, enc_hbm, ctx_hbm,
                   h1_ref, c1_ref, cs_ref, pg_ref,
                   emb_s, enc_s, ctx_s, sem_emb, sem_enc, sem_ctx):
    c = pl.program_id(0)
    b0 = c * BB

    enc_cp = pltpu.make_async_copy(enc_hbm.at[pl.ds(b0, BB)], enc_s, sem_enc)
    enc_cp.start()
    ctx_cp = pltpu.make_async_copy(ctx_hbm.at[pl.ds(b0, BB)], ctx_s, sem_ctx)
    ctx_cp.start()
    for mi in range(BB):
        idx = ids_ref[b0 + mi]
        pltpu.make_async_copy(emb_hbm.at[idx], emb_s.at[pl.ds(mi * 4, 4)],
                              sem_emb).start()

    h0 = h0_ref[...]
    c0 = c0_ref[...]
    dn = (((1,), (1,)), ((), ()))
    gates = lax.dot_general(h0, whh_ref[...], dn,
                            preferred_element_type=jnp.float32)

    for mi in range(BB):
        pltpu.make_async_copy(emb_hbm.at[0], emb_s.at[pl.ds(0, 4)],
                              sem_emb).wait()
    emb_chunks = [emb_s[cc::4, :] for cc in range(4)]
    for cc in range(4):
        gates = gates + lax.dot_general(
            emb_chunks[cc], wih_ref[:, cc * 128:(cc + 1) * 128], dn,
            preferred_element_type=jnp.float32)
    gates = gates + bihh_ref[...]

    ii = gates[:, 0:H]
    ff = gates[:, H:2 * H]
    gg = gates[:, 2 * H:3 * H]
    oo = gates[:, 3 * H:4 * H]
    c1 = jax.nn.sigmoid(ff) * c0 + jax.nn.sigmoid(ii) * jnp.tanh(gg)
    h1 = jax.nn.sigmoid(oo) * jnp.tanh(c1)
    h1_ref[...] = h1
    c1_ref[...] = c1

    def attend(mem_s, w_ref, b_ref, cp):
        dec = lax.dot_general(h1, w_ref[...], dn,
                              preferred_element_type=jnp.float32) + b_ref[...]
        cp.wait()
        cols = []
        for tc in range(0, TENC, 8):
            chunk = mem_s[:, tc:tc + 8, :]
            cols.append(jnp.sum(chunk * dec[:, None, :], axis=2))
        logits = jnp.concatenate(cols, axis=1)
        m = jnp.max(logits, axis=1, keepdims=True)
        e = jnp.exp(logits - m)
        sc = e / jnp.sum(e, axis=1, keepdims=True)
        ctxv = jnp.zeros((BB, H), jnp.float32)
        for tc in range(0, TENC, 8):
            chunk = mem_s[:, tc:tc + 8, :]
            ctxv = ctxv + jnp.sum(chunk * sc[:, tc:tc + 8, None], axis=1)
        return sc, ctxv

    _, context = attend(enc_s, attnw_ref, attnb_ref, enc_cp)
    sc2, ctx_context = attend(ctx_s, cattnw_ref, cattnb_ref, ctx_cp)
    cs_ref[...] = sc2

    gw = genw_ref[...]
    pls = (jnp.sum(context * gw[:, 0:H], axis=1, keepdims=True)
           + jnp.sum(ctx_context * gw[:, H:2 * H], axis=1, keepdims=True)
           + jnp.sum(h1 * gw[:, 2 * H:3 * H], axis=1, keepdims=True))
    for cc in range(4):
        lo = 3 * H + cc * 128
        pls = pls + jnp.sum(emb_chunks[cc] * gw[:, lo:lo + 128], axis=1,
                            keepdims=True)
    pg = jax.nn.sigmoid(pls + genb_ref[0])
    ctx_len = jnp.sum((ctxvar_ref[...] > 0).astype(jnp.float32), axis=1,
                      keepdims=True)
    pg = jnp.where(ctx_len == 0.0, 1.0, pg)
    pg_ref[...] = jnp.broadcast_to(pg, (BB, 128))


def _stage2_kernel(hs_ref, pg_ref, cs_smem, var_smem, w_ref, b_ref,
                   out_ref, logits_s, pcopy_s, ms_ref, ss_ref):
    c = pl.program_id(0)
    p = pl.program_id(1)
    j = pl.program_id(2)

    @pl.when(jnp.logical_and(p == 0, j == 0))
    def _init():
        ms_ref[...] = jnp.full_like(ms_ref, NEG)
        ss_ref[...] = jnp.zeros_like(ss_ref)
        pcopy_s[...] = jnp.zeros_like(pcopy_s)
        iota_s = lax.broadcasted_iota(jnp.int32, (8, BB), 0)
        iota_l = lax.broadcasted_iota(jnp.int32, (8, BB), 1)
        for b in range(BB):
            lane_mask = iota_l == b
            row = c * BB + b

            def body(t, _, lane_mask=lane_mask, row=row):
                idx = var_smem[row, t]
                s = cs_smem[row, t]
                base = pl.multiple_of((idx >> 3) << 3, 8)
                chunk = pcopy_s[pl.ds(base, 8), :]
                upd = jnp.where(
                    jnp.logical_and(iota_s == (idx & 7), lane_mask), s, 0.0)
                pcopy_s[pl.ds(base, 8), :] = chunk + upd
                return 0

            lax.fori_loop(0, TCTX, body, 0)

    @pl.when(p == 0)
    def _logits():
        lt = lax.dot_general(hs_ref[...], w_ref[...], (((1,), (1,)), ((), ())),
                             preferred_element_type=jnp.float32) + b_ref[0]
        gcol = j * TV + lax.broadcasted_iota(jnp.int32, (1, TV), 1)
        lt = jnp.where(gcol < V, lt, NEG)
        logits_s[pl.ds(j, 1)] = lt[None]
        m_old = ms_ref[:, 0:1]
        m_new = jnp.maximum(m_old, jnp.max(lt, axis=1, keepdims=True))
        s_new = (ss_ref[:, 0:1] * jnp.exp(m_old - m_new)
                 + jnp.sum(jnp.exp(lt - m_new), axis=1, keepdims=True))
        ms_ref[:, 0:1] = m_new
        ss_ref[:, 0:1] = s_new

    @pl.when(p == 1)
    def _final():
        lt = logits_s[pl.ds(j, 1)][0]
        m = ms_ref[:, 0:1]
        s = ss_ref[:, 0:1]
        pv = jnp.exp(lt - m) / s
        pc = pcopy_s[pl.ds(j * TV, TV), :]
        pct = pc.T
        pg = pg_ref[:, 0:1]
        prob = pg * pv + (1.0 - pg) * pct
        out_ref[...] = jnp.log(jnp.maximum(prob, 1e-10))


def kernel(input_ids, h0, c0, encoder_outputs, context_type_outputs,
           context_type_variable, embedding, W_ih, W_hh, b_ih, b_hh,
           attn_W, attn_b, ctx_attn_W, ctx_attn_b, gen_W, gen_b,
           sig_bias, out_W, out_b):
    ids = input_ids[:, 0].astype(jnp.int32)
    emb3 = embedding.reshape(V, 4, 128)
    bihh = (b_ih + b_hh).reshape(1, 4 * H)
    ab = attn_b.reshape(1, H)
    cab = ctx_attn_b.reshape(1, H)
    gb = (gen_b + sig_bias).reshape(1)
    ctxvar = context_type_variable.astype(jnp.int32)

    h1, c1, cscores, pgen = pl.pallas_call(
        _stage1_kernel,
        grid=(NC,),
        in_specs=[
            pl.BlockSpec(memory_space=pltpu.SMEM),
            pl.BlockSpec((BB, H), lambda c: (c, 0)),
            pl.BlockSpec((BB, H), lambda c: (c, 0)),
            pl.BlockSpec((BB, TCTX), lambda c: (c, 0)),
            pl.BlockSpec((4 * H, E), lambda c: (0, 0)),
            pl.BlockSpec((4 * H, H), lambda c: (0, 0)),
            pl.BlockSpec((1, 4 * H), lambda c: (0, 0)),
            pl.BlockSpec((H, H), lambda c: (0, 0)),
            pl.BlockSpec((1, H), lambda c: (0, 0)),
            pl.BlockSpec((H, H), lambda c: (0, 0)),
            pl.BlockSpec((1, H), lambda c: (0, 0)),
            pl.BlockSpec((1, 3 * H + E), lambda c: (0, 0)),
            pl.BlockSpec(memory_space=pltpu.SMEM),
            pl.BlockSpec(memory_space=pl.ANY),
            pl.BlockSpec(memory_space=pl.ANY),
            pl.BlockSpec(memory_space=pl.ANY),
        ],
        out_specs=[
            pl.BlockSpec((BB, H), lambda c: (c, 0)),
            pl.BlockSpec((BB, H), lambda c: (c, 0)),
            pl.BlockSpec((BB, TCTX), lambda c: (c, 0)),
            pl.BlockSpec((BB, 128), lambda c: (c, 0)),
        ],
        out_shape=[
            jax.ShapeDtypeStruct((B, H), jnp.float32),
            jax.ShapeDtypeStruct((B, H), jnp.float32),
            jax.ShapeDtypeStruct((B, TCTX), jnp.float32),
            jax.ShapeDtypeStruct((B, 128), jnp.float32),
        ],
        scratch_shapes=[
            pltpu.VMEM((BB * 4, 128), jnp.float32),
            pltpu.VMEM((BB, TENC, H), jnp.float32),
            pltpu.VMEM((BB, TCTX, H), jnp.float32),
            pltpu.SemaphoreType.DMA,
            pltpu.SemaphoreType.DMA,
            pltpu.SemaphoreType.DMA,
        ],
        compiler_params=pltpu.CompilerParams(
            dimension_semantics=("parallel",),
            vmem_limit_bytes=56 * 1024 * 1024,
        ),
    )(ids, h0[0], c0[0], ctxvar, W_ih, W_hh, bihh, attn_W, ab,
      ctx_attn_W, cab, gen_W, gb, emb3, encoder_outputs,
      context_type_outputs)

    logp = pl.pallas_call(
        _stage2_kernel,
        grid=(NC, 2, NV),
        in_specs=[
            pl.BlockSpec((BB, H), lambda c, p, j: (c, 0)),
            pl.BlockSpec((BB, 128), lambda c, p, j: (c, 0)),
            pl.BlockSpec(memory_space=pltpu.SMEM),
            pl.BlockSpec(memory_space=pltpu.SMEM),
            pl.BlockSpec((TV, H),
                         lambda c, p, j: (jnp.where(p == 0, j, NV - 1), 0)),
            pl.BlockSpec((1, 1, TV),
                         lambda c, p, j: (jnp.where(p == 0, j, NV - 1), 0, 0)),
        ],
        out_specs=pl.BlockSpec(
            (BB, TV), lambda c, p, j: (c, jnp.where(p == 1, j, 0))),
        out_shape=jax.ShapeDtypeStruct((B, VOUT), jnp.float32),
        scratch_shapes=[
            pltpu.VMEM((NV, BB, TV), jnp.float32),
            pltpu.VMEM((WTOT, BB), jnp.float32),
            pltpu.VMEM((BB, 128), jnp.float32),
            pltpu.VMEM((BB, 128), jnp.float32),
        ],
        compiler_params=pltpu.CompilerParams(
            dimension_semantics=("parallel", "arbitrary", "arbitrary"),
            vmem_limit_bytes=56 * 1024 * 1024,
        ),
    )(h1, pgen, cscores, ctxvar, out_W,
      jnp.pad(out_b, (0, WTOT - V)).reshape(NV, 1, TV))

    return logp, h1[None], c1[None]


# speedup vs baseline: 1.1201x; 1.1201x over previous
import jax
import jax.numpy as jnp
from jax import lax
from jax.experimental import pallas as pl
from jax.experimental.pallas import tpu as pltpu

V, E, H, B = 50000, 512, 512, 128
TENC, TCTX, OOV = 128, 128, 50
BB = 64
NC = B // BB
TV = 512
NV = (V + TV - 1) // TV
WTOT = NV * TV
VOUT = V + OOV
NHALF = NV // 2
CHALF = NHALF * TV
NENT = B * TCTX
NL = NENT + 128
NEG = -1e30


def _stage1_kernel(ids_ref, h0_ref, c0_ref, ctxvar_ref,
                   wih_ref, whh_ref, bihh_ref, attnw_ref, attnb_ref,
                   cattnw_ref, cattnb_ref, genw_ref, genb_ref,
                   emb_hbm, enc_hbm, ctx_hbm,
                   h1_ref, c1_ref, cs_ref, pg_ref,
                   emb_s, enc_s, ctx_s, sem_emb, sem_enc, sem_ctx):
    c = pl.program_id(0)
    b0 = c * BB

    enc_cp = pltpu.make_async_copy(enc_hbm.at[pl.ds(b0, BB)], enc_s, sem_enc)
    enc_cp.start()
    ctx_cp = pltpu.make_async_copy(ctx_hbm.at[pl.ds(b0, BB)], ctx_s, sem_ctx)
    ctx_cp.start()
    for mi in range(BB):
        idx = ids_ref[b0 + mi]
        pltpu.make_async_copy(emb_hbm.at[idx], emb_s.at[pl.ds(mi * 4, 4)],
                              sem_emb).start()

    h0 = h0_ref[...]
    c0 = c0_ref[...]
    dn = (((1,), (1,)), ((), ()))
    gates = lax.dot_general(h0, whh_ref[...], dn,
                            preferred_element_type=jnp.float32)

    for mi in range(BB):
        pltpu.make_async_copy(emb_hbm.at[0], emb_s.at[pl.ds(0, 4)],
                              sem_emb).wait()
    emb_chunks = [emb_s[cc::4, :] for cc in range(4)]
    for cc in range(4):
        gates = gates + lax.dot_general(
            emb_chunks[cc], wih_ref[:, cc * 128:(cc + 1) * 128], dn,
            preferred_element_type=jnp.float32)
    gates = gates + bihh_ref[...]

    ii = gates[:, 0:H]
    ff = gates[:, H:2 * H]
    gg = gates[:, 2 * H:3 * H]
    oo = gates[:, 3 * H:4 * H]
    c1 = jax.nn.sigmoid(ff) * c0 + jax.nn.sigmoid(ii) * jnp.tanh(gg)
    h1 = jax.nn.sigmoid(oo) * jnp.tanh(c1)
    h1_ref[...] = h1
    c1_ref[...] = c1

    def attend(mem_s, w_ref, b_ref, cp):
        dec = lax.dot_general(h1, w_ref[...], dn,
                              preferred_element_type=jnp.float32) + b_ref[...]
        cp.wait()
        cols = []
        for tc in range(0, TENC, 8):
            chunk = mem_s[:, tc:tc + 8, :]
            cols.append(jnp.sum(chunk * dec[:, None, :], axis=2))
        logits = jnp.concatenate(cols, axis=1)
        m = jnp.max(logits, axis=1, keepdims=True)
        e = jnp.exp(logits - m)
        sc = e / jnp.sum(e, axis=1, keepdims=True)
        ctxv = jnp.zeros((BB, H), jnp.float32)
        for tc in range(0, TENC, 8):
            chunk = mem_s[:, tc:tc + 8, :]
            ctxv = ctxv + jnp.sum(chunk * sc[:, tc:tc + 8, None], axis=1)
        return sc, ctxv

    _, context = attend(enc_s, attnw_ref, attnb_ref, enc_cp)
    sc2, ctx_context = attend(ctx_s, cattnw_ref, cattnb_ref, ctx_cp)
    cs_ref[...] = sc2

    gw = genw_ref[...]
    pls = (jnp.sum(context * gw[:, 0:H], axis=1, keepdims=True)
           + jnp.sum(ctx_context * gw[:, H:2 * H], axis=1, keepdims=True)
           + jnp.sum(h1 * gw[:, 2 * H:3 * H], axis=1, keepdims=True))
    for cc in range(4):
        lo = 3 * H + cc * 128
        pls = pls + jnp.sum(emb_chunks[cc] * gw[:, lo:lo + 128], axis=1,
                            keepdims=True)
    pg = jax.nn.sigmoid(pls + genb_ref[0])
    ctx_len = jnp.sum((ctxvar_ref[...] > 0).astype(jnp.float32), axis=1,
                      keepdims=True)
    pg = jnp.where(ctx_len == 0.0, 1.0, pg)
    pg_ref[...] = jnp.broadcast_to(pg, (BB, 128))


def _stage2_kernel(hs_ref, pg_ref, wa_ref, wb_ref, cs1_ref, meta_ref,
                   w_ref, b_ref,
                   out_ref, logits_s, pca_s, pcb_s, ms_ref, ss_ref):
    p = pl.program_id(0)
    j = pl.program_id(1)

    @pl.when(jnp.logical_and(p == 0, j == 0))
    def _init():
        ms_ref[...] = jnp.full_like(ms_ref, NEG)
        ss_ref[...] = jnp.zeros_like(ss_ref)
        pca_s[...] = jnp.zeros_like(pca_s)
        pcb_s[...] = jnp.zeros_like(pcb_s)
        iota_s = lax.broadcasted_iota(jnp.int32, (8, B), 0)
        iota_l = lax.broadcasted_iota(jnp.int32, (8, B), 1)
        npairs = meta_ref[0]

        def sbody(o, _):
            for u in range(4):
                k = o * 4 + u
                for wref, buf in ((wa_ref, pca_s), (wb_ref, pcb_s)):
                    w = wref[k]
                    idx = w & 0x7FFF
                    pe = w >> 16
                    s = cs1_ref[pe]
                    b = pe >> 7
                    base = pl.multiple_of((idx >> 3) << 3, 8)
                    chunk = buf[pl.ds(base, 8), :]
                    upd = jnp.where(
                        jnp.logical_and(iota_s == (idx & 7), iota_l == b),
                        s, 0.0)
                    buf[pl.ds(base, 8), :] = chunk + upd
            return 0

        lax.fori_loop(0, npairs, sbody, 0)

    @pl.when(p == 0)
    def _logits():
        lt = lax.dot_general(hs_ref[...], w_ref[...], (((1,), (1,)), ((), ())),
                             preferred_element_type=jnp.float32) + b_ref[0]
        gcol = j * TV + lax.broadcasted_iota(jnp.int32, (1, TV), 1)
        lt = jnp.where(gcol < V, lt, NEG)
        logits_s[pl.ds(j, 1)] = lt[None]
        m_old = ms_ref[:, 0:1]
        m_new = jnp.maximum(m_old, jnp.max(lt, axis=1, keepdims=True))
        s_new = (ss_ref[:, 0:1] * jnp.exp(m_old - m_new)
                 + jnp.sum(jnp.exp(lt - m_new), axis=1, keepdims=True))
        ms_ref[:, 0:1] = m_new
        ss_ref[:, 0:1] = s_new

    @pl.when(p == 1)
    def _final():
        lt = logits_s[pl.ds(j, 1)][0]
        m = ms_ref[:, 0:1]
        s = ss_ref[:, 0:1]
        pv = jnp.exp(lt - m) / s
        ja = jnp.minimum(j, NHALF - 1) * TV
        jb = jnp.maximum(j - NHALF, 0) * TV
        pca = pca_s[pl.ds(ja, TV), :]
        pcb = pcb_s[pl.ds(jb, TV), :]
        pc = jnp.where(j < NHALF, pca, pcb)
        pct = pc.T
        pg = pg_ref[:, 0:1]
        prob = pg * pv + (1.0 - pg) * pct
        out_ref[...] = jnp.log(jnp.maximum(prob, 1e-10))


def kernel(input_ids, h0, c0, encoder_outputs, context_type_outputs,
           context_type_variable, embedding, W_ih, W_hh, b_ih, b_hh,
           attn_W, attn_b, ctx_attn_W, ctx_attn_b, gen_W, gen_b,
           sig_bias, out_W, out_b):
    ids = input_ids[:, 0].astype(jnp.int32)
    emb3 = embedding.reshape(V, 4, 128)
    bihh = (b_ih + b_hh).reshape(1, 4 * H)
    ab = attn_b.reshape(1, H)
    cab = ctx_attn_b.reshape(1, H)
    gb = (gen_b + sig_bias).reshape(1)
    ctxvar = context_type_variable.astype(jnp.int32)

    h1, c1, cscores, pgen = pl.pallas_call(
        _stage1_kernel,
        grid=(NC,),
        in_specs=[
            pl.BlockSpec(memory_space=pltpu.SMEM),
            pl.BlockSpec((BB, H), lambda c: (c, 0)),
            pl.BlockSpec((BB, H), lambda c: (c, 0)),
            pl.BlockSpec((BB, TCTX), lambda c: (c, 0)),
            pl.BlockSpec((4 * H, E), lambda c: (0, 0)),
            pl.BlockSpec((4 * H, H), lambda c: (0, 0)),
            pl.BlockSpec((1, 4 * H), lambda c: (0, 0)),
            pl.BlockSpec((H, H), lambda c: (0, 0)),
            pl.BlockSpec((1, H), lambda c: (0, 0)),
            pl.BlockSpec((H, H), lambda c: (0, 0)),
            pl.BlockSpec((1, H), lambda c: (0, 0)),
            pl.BlockSpec((1, 3 * H + E), lambda c: (0, 0)),
            pl.BlockSpec(memory_space=pltpu.SMEM),
            pl.BlockSpec(memory_space=pl.ANY),
            pl.BlockSpec(memory_space=pl.ANY),
            pl.BlockSpec(memory_space=pl.ANY),
        ],
        out_specs=[
            pl.BlockSpec((BB, H), lambda c: (c, 0)),
            pl.BlockSpec((BB, H), lambda c: (c, 0)),
            pl.BlockSpec((BB, TCTX), lambda c: (c, 0)),
            pl.BlockSpec((BB, 128), lambda c: (c, 0)),
        ],
        out_shape=[
            jax.ShapeDtypeStruct((B, H), jnp.float32),
            jax.ShapeDtypeStruct((B, H), jnp.float32),
            jax.ShapeDtypeStruct((B, TCTX), jnp.float32),
            jax.ShapeDtypeStruct((B, 128), jnp.float32),
        ],
        scratch_shapes=[
            pltpu.VMEM((BB * 4, 128), jnp.float32),
            pltpu.VMEM((BB, TENC, H), jnp.float32),
            pltpu.VMEM((BB, TCTX, H), jnp.float32),
            pltpu.SemaphoreType.DMA,
            pltpu.SemaphoreType.DMA,
            pltpu.SemaphoreType.DMA,
        ],
        compiler_params=pltpu.CompilerParams(
            dimension_semantics=("arbitrary",),
            vmem_limit_bytes=56 * 1024 * 1024,
        ),
    )(ids, h0[0], c0[0], ctxvar, W_ih, W_hh, bihh, attn_W, ab,
      ctx_attn_W, cab, gen_W, gb, emb3, encoder_outputs,
      context_type_outputs)

    flat = ctxvar.reshape(-1)
    in_b = flat >= CHALF
    perm = jnp.argsort(in_b.astype(jnp.int32), stable=True).astype(jnp.int32)
    n_a = (NENT - in_b.sum()).astype(jnp.int32)
    n_b = NENT - n_a
    ar = jnp.arange(NL, dtype=jnp.int32)
    pa = jnp.where(ar < n_a, perm[jnp.clip(ar, 0, NENT - 1)], NENT)
    ia = jnp.where(ar < n_a, flat[jnp.clip(pa, 0, NENT - 1)], 0)
    wa = ia | (pa << 16)
    pb = jnp.where(ar < n_b, perm[jnp.clip(ar + n_a, 0, NENT - 1)], NENT)
    ib = jnp.where(ar < n_b, flat[jnp.clip(pb, 0, NENT - 1)] - CHALF, 0)
    wb = ib | (pb << 16)
    cs1 = jnp.concatenate([cscores.reshape(-1), jnp.zeros((1,), jnp.float32)])
    npairs = (jnp.maximum(n_a, n_b) + 3) // 4
    meta = npairs.reshape(1)

    logp = pl.pallas_call(
        _stage2_kernel,
        grid=(2, NV),
        in_specs=[
            pl.BlockSpec((B, H), lambda p, j: (0, 0)),
            pl.BlockSpec((B, 128), lambda p, j: (0, 0)),
            pl.BlockSpec(memory_space=pltpu.SMEM),
            pl.BlockSpec(memory_space=pltpu.SMEM),
            pl.BlockSpec(memory_space=pltpu.SMEM),
            pl.BlockSpec(memory_space=pltpu.SMEM),
            pl.BlockSpec((TV, H),
                         lambda p, j: (jnp.where(p == 0, j, NV - 1), 0)),
            pl.BlockSpec((1, 1, TV),
                         lambda p, j: (jnp.where(p == 0, j, NV - 1), 0, 0)),
        ],
        out_specs=pl.BlockSpec(
            (B, TV), lambda p, j: (0, jnp.where(p == 1, j, 0))),
        out_shape=jax.ShapeDtypeStruct((B, VOUT), jnp.float32),
        scratch_shapes=[
            pltpu.VMEM((NV, B, TV), jnp.float32),
            pltpu.VMEM((CHALF, B), jnp.float32),
            pltpu.VMEM((CHALF, B), jnp.float32),
            pltpu.VMEM((B, 128), jnp.float32),
            pltpu.VMEM((B, 128), jnp.float32),
        ],
        compiler_params=pltpu.CompilerParams(
            dimension_semantics=("arbitrary", "arbitrary"),
            vmem_limit_bytes=58 * 1024 * 1024,
        ),
    )(h1, pgen, wa, wb, cs1, meta, out_W,
      jnp.pad(out_b, (0, WTOT - V)).reshape(NV, 1, TV))

    return logp, h1[None], c1[None]


# speedup vs baseline: 1.9480x; 1.7391x over previous
import jax
import jax.numpy as jnp
from jax import lax
from jax.experimental import pallas as pl
from jax.experimental.pallas import tpu as pltpu

V, E, H, B = 50000, 512, 512, 128
TENC, TCTX, OOV = 128, 128, 50
BB = 64
NC = B // BB
TV = 512
NV = (V + TV - 1) // TV
WTOT = NV * TV
VOUT = V + OOV
NHALF = NV // 2
CHALF = NHALF * TV
NENT = B * TCTX
NL = NENT + 128
NEG = -1e30


def _stage1_kernel(ids_ref, h0_ref, c0_ref, ctxvar_ref,
                   wih_ref, whh_ref, bihh_ref, attnw_ref, attnb_ref,
                   cattnw_ref, cattnb_ref, genw_ref, genb_ref,
                   emb_hbm, enc_hbm, ctx_hbm,
                   h1_ref, c1_ref, cs_ref, pg_ref,
                   emb_s, enc_s, ctx_s, sem_emb, sem_enc, sem_ctx):
    c = pl.program_id(0)
    b0 = c * BB

    enc_cp = pltpu.make_async_copy(enc_hbm.at[pl.ds(b0, BB)], enc_s, sem_enc)
    enc_cp.start()
    ctx_cp = pltpu.make_async_copy(ctx_hbm.at[pl.ds(b0, BB)], ctx_s, sem_ctx)
    ctx_cp.start()
    for mi in range(BB):
        idx = ids_ref[b0 + mi]
        pltpu.make_async_copy(emb_hbm.at[pl.ds(idx, 1)],
                              emb_s.at[pl.ds(mi, 1)], sem_emb).start()

    h0 = h0_ref[...]
    c0 = c0_ref[...]
    dn = (((1,), (1,)), ((), ()))
    gates = lax.dot_general(h0, whh_ref[...], dn,
                            preferred_element_type=jnp.float32)

    for mi in range(BB):
        pltpu.make_async_copy(emb_hbm.at[pl.ds(0, 1)],
                              emb_s.at[pl.ds(0, 1)], sem_emb).wait()
    emb = emb_s[...]
    gates = gates + lax.dot_general(emb, wih_ref[...], dn,
                                    preferred_element_type=jnp.float32)
    gates = gates + bihh_ref[...]

    ii = gates[:, 0:H]
    ff = gates[:, H:2 * H]
    gg = gates[:, 2 * H:3 * H]
    oo = gates[:, 3 * H:4 * H]
    c1 = jax.nn.sigmoid(ff) * c0 + jax.nn.sigmoid(ii) * jnp.tanh(gg)
    h1 = jax.nn.sigmoid(oo) * jnp.tanh(c1)
    h1_ref[...] = h1
    c1_ref[...] = c1

    def attend(mem_s, w_ref, b_ref, cp):
        dec = lax.dot_general(h1, w_ref[...], dn,
                              preferred_element_type=jnp.float32) + b_ref[...]
        cp.wait()
        cols = []
        for tc in range(0, TENC, 8):
            chunk = mem_s[:, tc:tc + 8, :]
            cols.append(jnp.sum(chunk * dec[:, None, :], axis=2))
        logits = jnp.concatenate(cols, axis=1)
        m = jnp.max(logits, axis=1, keepdims=True)
        e = jnp.exp(logits - m)
        sc = e / jnp.sum(e, axis=1, keepdims=True)
        ctxv = jnp.zeros((BB, H), jnp.float32)
        for tc in range(0, TENC, 8):
            chunk = mem_s[:, tc:tc + 8, :]
            ctxv = ctxv + jnp.sum(chunk * sc[:, tc:tc + 8, None], axis=1)
        return sc, ctxv

    _, context = attend(enc_s, attnw_ref, attnb_ref, enc_cp)
    sc2, ctx_context = attend(ctx_s, cattnw_ref, cattnb_ref, ctx_cp)
    cs_ref[...] = sc2

    gw = genw_ref[...]
    pls = (jnp.sum(context * gw[:, 0:H], axis=1, keepdims=True)
           + jnp.sum(ctx_context * gw[:, H:2 * H], axis=1, keepdims=True)
           + jnp.sum(h1 * gw[:, 2 * H:3 * H], axis=1, keepdims=True))
    pls = pls + jnp.sum(emb * gw[:, 3 * H:3 * H + E], axis=1, keepdims=True)
    pg = jax.nn.sigmoid(pls + genb_ref[0])
    ctx_len = jnp.sum((ctxvar_ref[...] > 0).astype(jnp.float32), axis=1,
                      keepdims=True)
    pg = jnp.where(ctx_len == 0.0, 1.0, pg)
    pg_ref[...] = jnp.broadcast_to(pg, (BB, 128))


def _stage2_kernel(hs_ref, pg_ref, cs_smem, var_smem,
                   w_ref, b_ref,
                   out_ref, logits_s, pca_s, ms_ref, ss_ref):
    p = pl.program_id(0)
    j = pl.program_id(1)

    @pl.when(jnp.logical_and(p == 0, j == 0))
    def _init():
        ms_ref[...] = jnp.full_like(ms_ref, NEG)
        ss_ref[...] = jnp.zeros_like(ss_ref)
        pca_s[...] = jnp.zeros_like(pca_s)
        iota_s = lax.broadcasted_iota(jnp.int32, (8, B), 0)
        iota_l = lax.broadcasted_iota(jnp.int32, (8, B), 1)

        def sbody(o, _):
            for u in range(4):
                k = o * 4 + u
                row = k >> 7
                t = k & 127
                idx = var_smem[row, t]
                s = cs_smem[row, t]
                base = pl.multiple_of((idx >> 3) << 3, 8)
                chunk = pca_s[pl.ds(base, 8), :]
                upd = jnp.where(
                    jnp.logical_and(iota_s == (idx & 7), iota_l == row),
                    s, 0.0)
                pca_s[pl.ds(base, 8), :] = chunk + upd
            return 0

        lax.fori_loop(0, NENT // 4, sbody, 0)

    @pl.when(p == 0)
    def _logits():
        lt = lax.dot_general(hs_ref[...], w_ref[...], (((1,), (1,)), ((), ())),
                             preferred_element_type=jnp.float32) + b_ref[0]
        gcol = j * TV + lax.broadcasted_iota(jnp.int32, (1, TV), 1)
        lt = jnp.where(gcol < V, lt, NEG)
        logits_s[pl.ds(j, 1)] = lt[None]
        m_old = ms_ref[:, 0:1]
        m_new = jnp.maximum(m_old, jnp.max(lt, axis=1, keepdims=True))
        s_new = (ss_ref[:, 0:1] * jnp.exp(m_old - m_new)
                 + jnp.sum(jnp.exp(lt - m_new), axis=1, keepdims=True))
        ms_ref[:, 0:1] = m_new
        ss_ref[:, 0:1] = s_new

    @pl.when(p == 1)
    def _final():
        lt = logits_s[pl.ds(j, 1)][0]
        m = ms_ref[:, 0:1]
        s = ss_ref[:, 0:1]
        pv = jnp.exp(lt - m) / s
        pct = pca_s[pl.ds(j * TV, TV), :].T
        pg = pg_ref[:, 0:1]
        prob = pg * pv + (1.0 - pg) * pct
        out_ref[...] = jnp.log(jnp.maximum(prob, 1e-10))


def kernel(input_ids, h0, c0, encoder_outputs, context_type_outputs,
           context_type_variable, embedding, W_ih, W_hh, b_ih, b_hh,
           attn_W, attn_b, ctx_attn_W, ctx_attn_b, gen_W, gen_b,
           sig_bias, out_W, out_b):
    ids = input_ids[:, 0].astype(jnp.int32)
    bihh = (b_ih + b_hh).reshape(1, 4 * H)
    ab = attn_b.reshape(1, H)
    cab = ctx_attn_b.reshape(1, H)
    gb = (gen_b + sig_bias).reshape(1)
    ctxvar = context_type_variable.astype(jnp.int32)

    h1, c1, cscores, pgen = pl.pallas_call(
        _stage1_kernel,
        grid=(NC,),
        in_specs=[
            pl.BlockSpec(memory_space=pltpu.SMEM),
            pl.BlockSpec((BB, H), lambda c: (c, 0)),
            pl.BlockSpec((BB, H), lambda c: (c, 0)),
            pl.BlockSpec((BB, TCTX), lambda c: (c, 0)),
            pl.BlockSpec((4 * H, E), lambda c: (0, 0)),
            pl.BlockSpec((4 * H, H), lambda c: (0, 0)),
            pl.BlockSpec((1, 4 * H), lambda c: (0, 0)),
            pl.BlockSpec((H, H), lambda c: (0, 0)),
            pl.BlockSpec((1, H), lambda c: (0, 0)),
            pl.BlockSpec((H, H), lambda c: (0, 0)),
            pl.BlockSpec((1, H), lambda c: (0, 0)),
            pl.BlockSpec((1, 3 * H + E), lambda c: (0, 0)),
            pl.BlockSpec(memory_space=pltpu.SMEM),
            pl.BlockSpec(memory_space=pl.ANY),
            pl.BlockSpec(memory_space=pl.ANY),
            pl.BlockSpec(memory_space=pl.ANY),
        ],
        out_specs=[
            pl.BlockSpec((BB, H), lambda c: (c, 0)),
            pl.BlockSpec((BB, H), lambda c: (c, 0)),
            pl.BlockSpec((BB, TCTX), lambda c: (c, 0)),
            pl.BlockSpec((BB, 128), lambda c: (c, 0)),
        ],
        out_shape=[
            jax.ShapeDtypeStruct((B, H), jnp.float32),
            jax.ShapeDtypeStruct((B, H), jnp.float32),
            jax.ShapeDtypeStruct((B, TCTX), jnp.float32),
            jax.ShapeDtypeStruct((B, 128), jnp.float32),
        ],
        scratch_shapes=[
            pltpu.VMEM((BB, E), jnp.float32),
            pltpu.VMEM((BB, TENC, H), jnp.float32),
            pltpu.VMEM((BB, TCTX, H), jnp.float32),
            pltpu.SemaphoreType.DMA,
            pltpu.SemaphoreType.DMA,
            pltpu.SemaphoreType.DMA,
        ],
        compiler_params=pltpu.CompilerParams(
            dimension_semantics=("arbitrary",),
            vmem_limit_bytes=56 * 1024 * 1024,
        ),
    )(ids, h0[0], c0[0], ctxvar, W_ih, W_hh, bihh, attn_W, ab,
      ctx_attn_W, cab, gen_W, gb, embedding, encoder_outputs,
      context_type_outputs)

    logp = pl.pallas_call(
        _stage2_kernel,
        grid=(2, NV),
        in_specs=[
            pl.BlockSpec((B, H), lambda p, j: (0, 0)),
            pl.BlockSpec((B, 128), lambda p, j: (0, 0)),
            pl.BlockSpec(memory_space=pltpu.SMEM),
            pl.BlockSpec(memory_space=pltpu.SMEM),
            pl.BlockSpec((TV, H),
                         lambda p, j: (jnp.where(p == 0, j, NV - 1), 0)),
            pl.BlockSpec((1, 1, TV),
                         lambda p, j: (jnp.where(p == 0, j, NV - 1), 0, 0)),
        ],
        out_specs=pl.BlockSpec(
            (B, TV), lambda p, j: (0, jnp.where(p == 1, j, 0))),
        out_shape=jax.ShapeDtypeStruct((B, VOUT), jnp.float32),
        scratch_shapes=[
            pltpu.VMEM((NV, B, TV), jnp.float32),
            pltpu.VMEM((WTOT, B), jnp.float32),
            pltpu.VMEM((B, 128), jnp.float32),
            pltpu.VMEM((B, 128), jnp.float32),
        ],
        compiler_params=pltpu.CompilerParams(
            dimension_semantics=("arbitrary", "arbitrary"),
            vmem_limit_bytes=58 * 1024 * 1024,
        ),
    )(h1, pgen, cscores, ctxvar, out_W,
      jnp.pad(out_b, (0, WTOT - V)).reshape(NV, 1, TV))

    return logp, h1[None], c1[None]
